# Initial kernel scaffold; baseline (speedup 1.0000x reference)
import jax
import jax.numpy as jnp
from jax.experimental import pallas as pl


def kernel(x_alpha, x_beta, x_theta, params, edge_index_alpha, edge_index_beta, edge_index_theta):
    raise NotImplementedError("write your pallas kernel here")


# speedup vs baseline: 30.6777x; 30.6777x over previous
import functools

import jax
import jax.numpy as jnp
from jax import lax
from jax.experimental import pallas as pl
from jax.experimental.pallas import tpu as pltpu
from jax.experimental.pallas import tpu_sc as plsc

N = 10000
NP1 = 10112
E = 320000
EE = E + N
NW = 32
EPT = 10368
EPAD = NW * EPT
ROWS_PT = NP1 // 16
MIN_DEN = 1e-16
BN_EPS = 1e-5


def _mm_body(x_ref, w_ref, wa_ref, oh_ref, h_ref, asd_ref):
    x = x_ref[:]
    h_ref[:] = jnp.dot(x, w_ref[:], preferred_element_type=jnp.float32) + oh_ref[:]
    asd_ref[:] = jnp.dot(x, wa_ref[:], preferred_element_type=jnp.float32).T


@functools.cache
def _mm_call(din, doutp):
    return pl.pallas_call(
        _mm_body,
        out_shape=(
            jax.ShapeDtypeStruct((NP1, doutp), jnp.float32),
            jax.ShapeDtypeStruct((2, NP1), jnp.float32),
        ),
    )


def _fin_body(nout, final, acc_ref, b_ref, g_ref, be_ref, y_ref):
    full = acc_ref[0] + acc_ref[1]
    z = full[0:N, 0:nout]
    dn = full[0:N, nout:nout + 1]
    z = z / (dn + MIN_DEN) + b_ref[:]
    mu = jnp.mean(z, axis=0, keepdims=True)
    var = jnp.mean((z - mu) ** 2, axis=0, keepdims=True)
    y = (z - mu) * jax.lax.rsqrt(var + BN_EPS) * g_ref[:] + be_ref[:]
    y = jnp.maximum(y, 0.0)
    if final:
        y_ref[:] = y
    else:
        y_ref[:] = jnp.concatenate(
            [y, jnp.zeros((NP1 - N, nout), jnp.float32)], axis=0)


@functools.cache
def _fin_call(doutp, final):
    nout = doutp - 16
    rows = N if final else NP1
    return pl.pallas_call(
        functools.partial(_fin_body, nout, final),
        out_shape=jax.ShapeDtypeStruct((rows, nout), jnp.float32),
    )


def _sc_body(doutp, chunk, h_hbm, asd_hbm, src_hbm, dst_hbm, acc_hbm,
             asv, adv, srcv, dstv, rows, pbuf, acc_sh, gsem):
    c = lax.axis_index("c")
    s = lax.axis_index("s")
    w = c * 16 + s
    nq = doutp // 16
    k = EPT // chunk

    pltpu.sync_copy(asd_hbm.at[0], asv)
    pltpu.sync_copy(asd_hbm.at[1], adv)

    def mbody(i, m):
        return jnp.maximum(m, asv[pl.ds(i * 16, 16)])
    m16 = lax.fori_loop(0, NP1 // 16, mbody, jnp.zeros((16,), jnp.float32))
    gdn = lax.GatherDimensionNumbers(
        offset_dims=(), collapsed_slice_dims=(0,), start_index_map=(0,))
    for sh in (8, 4, 2, 1):
        idx = (jnp.arange(16, dtype=jnp.int32) + sh) % 16
        perm = lax.gather(m16, idx[:, None], gdn, slice_sizes=(1,),
                          mode=lax.GatherScatterMode.PROMISE_IN_BOUNDS)
        m16 = jnp.maximum(m16, perm)
    M = m16

    zeros16 = jnp.zeros((16,), jnp.float32)

    def zbody(i, carry):
        for q in range(nq):
            rows[i, pl.ds(q * 16, 16)] = zeros16
        return carry
    lax.fori_loop(0, chunk, zbody, 0)

    base = s * ROWS_PT
    for t in range(ROWS_PT // chunk):
        pltpu.sync_copy(rows, acc_sh.at[pl.ds(base + t * chunk, chunk)])
    rem = ROWS_PT % chunk
    if rem:
        pltpu.sync_copy(rows.at[pl.ds(0, rem)],
                        acc_sh.at[pl.ds(base + ROWS_PT - rem, rem)])
    plsc.subcore_barrier()

    def chunk_body(j, carry):
        pltpu.sync_copy(src_hbm.at[w, j], srcv.at[0])
        pltpu.sync_copy(dst_hbm.at[w, j], dstv.at[0])
        pltpu.async_copy(h_hbm.at[srcv.at[0]], rows, gsem).wait()
        for g in range(chunk // 16):
            sidx = srcv[0, pl.ds(g * 16, 16)]
            didx = dstv[0, pl.ds(g * 16, 16)]
            u = plsc.load_gather(asv, [sidx])
            v = plsc.load_gather(adv, [didx])
            t = u + v
            e = jnp.maximum(t, 0.2 * t)
            cm = M + v
            cb = jnp.maximum(cm, 0.2 * cm)
            pbuf[pl.ds(g * 16, 16)] = jnp.exp(e - cb)

        def scale_body(g, cc):
            p16 = pbuf[pl.ds(g * 16, 16)]
            for l in range(16):
                pv = jnp.full((16,), p16[l], jnp.float32)
                i = g * 16 + l
                for q in range(nq):
                    rows[i, pl.ds(q * 16, 16)] = rows[i, pl.ds(q * 16, 16)] * pv
            return cc
        lax.fori_loop(0, chunk // 16, scale_body, 0)
        pltpu.sync_copy(rows, acc_sh.at[dstv.at[0]], add=True)
        return carry
    lax.fori_loop(0, k, chunk_body, 0)

    plsc.subcore_barrier()
    pltpu.sync_copy(acc_sh.at[pl.ds(base, ROWS_PT)],
                    acc_hbm.at[c, pl.ds(base, ROWS_PT)])


@functools.cache
def _sc_call(doutp, chunk):
    mesh = plsc.VectorSubcoreMesh(core_axis_name="c", subcore_axis_name="s")
    k = EPT // chunk
    return pl.kernel(
        functools.partial(_sc_body, doutp, chunk),
        mesh=mesh,
        compiler_params=pltpu.CompilerParams(
            needs_layout_passes=False, use_tc_tiling_on_sc=False),
        out_type=jax.ShapeDtypeStruct((2, NP1, doutp), jnp.float32),
        scratch_types=[
            pltpu.VMEM((NP1,), jnp.float32),
            pltpu.VMEM((NP1,), jnp.float32),
            pltpu.VMEM((1, chunk), jnp.int32),
            pltpu.VMEM((1, chunk), jnp.int32),
            pltpu.VMEM((chunk, doutp), jnp.float32),
            pltpu.VMEM((chunk,), jnp.float32),
            pltpu.VMEM_SHARED((NP1, doutp), jnp.float32),
            pltpu.SemaphoreType.DMA,
        ],
    )


def _prep_edges(edge_index, chunk):
    src, dst = edge_index[0], edge_index[1]
    loops = jnp.arange(N, dtype=src.dtype)
    pad = EPAD - EE
    pad_src = jnp.arange(pad, dtype=src.dtype) % N
    pad_dst = N + (jnp.arange(pad, dtype=src.dtype) % (NP1 - N))
    k = EPT // chunk
    src = jnp.concatenate([src, loops, pad_src]).reshape(NW, k, chunk)
    dst = jnp.concatenate([dst, loops, pad_dst]).reshape(NW, k, chunk)
    return src, dst


def _band(x, edge_index, layers):
    x = jnp.pad(x, ((0, NP1 - N), (0, 0)))
    for li, p in enumerate(layers):
        din, dout = p["W"].shape
        doutp = dout + 16
        chunk = 64 if doutp > 96 else 128
        src3, dst3 = _prep_edges(edge_index, chunk)
        w_ext = jnp.pad(p["W"], ((0, 0), (0, 16)))
        marker = jnp.zeros((1, doutp), jnp.float32).at[0, dout].set(1.0)
        wa = jnp.stack([p["W"] @ p["a_src"], p["W"] @ p["a_dst"]], axis=1)
        h_ext, asd = _mm_call(din, doutp)(x, w_ext, wa, marker)
        acc = _sc_call(doutp, chunk)(h_ext, asd, src3, dst3)
        final = li == len(layers) - 1
        x = _fin_call(doutp, final)(
            acc, p["b"][None, :], p["gamma"][None, :], p["beta"][None, :])
    return x


def kernel(x_alpha, x_beta, x_theta, params,
           edge_index_alpha, edge_index_beta, edge_index_theta):
    z_a = _band(x_alpha, edge_index_alpha, params["alpha"])
    z_b = _band(x_beta, edge_index_beta, params["beta"])
    z_t = _band(x_theta, edge_index_theta, params["theta"])
    return (z_a, z_b, z_t)


# speedup vs baseline: 41.5848x; 1.3555x over previous
import functools

import jax
import jax.numpy as jnp
from jax import lax
from jax.experimental import pallas as pl
from jax.experimental.pallas import tpu as pltpu
from jax.experimental.pallas import tpu_sc as plsc

N = 10000
NP1 = 10112
E = 320000
EE = E + N
NW = 32
EPT = 10368
EPAD = NW * EPT
ROWS_PT = NP1 // 16
MIN_DEN = 1e-16
BN_EPS = 1e-5


def _mm_body(x_ref, w_ref, wa_ref, h_ref, asd_ref):
    x = x_ref[:]
    h_ref[:] = jnp.dot(x, w_ref[:], preferred_element_type=jnp.float32)
    asd_ref[:] = jnp.dot(x, wa_ref[:], preferred_element_type=jnp.float32).T


@functools.cache
def _mm_call(din, dout):
    return pl.pallas_call(
        _mm_body,
        out_shape=(
            jax.ShapeDtypeStruct((NP1, dout), jnp.float32),
            jax.ShapeDtypeStruct((2, NP1), jnp.float32),
        ),
    )


def _fin_body(nout, final, acc_ref, dn_ref, b_ref, g_ref, be_ref, y_ref):
    z = (acc_ref[0] + acc_ref[1])[0:N]
    dn = jnp.sum(dn_ref[0] + dn_ref[1], axis=0)[0:N]
    z = z / (dn[:, None] + MIN_DEN) + b_ref[:]
    mu = jnp.mean(z, axis=0, keepdims=True)
    var = jnp.mean((z - mu) ** 2, axis=0, keepdims=True)
    y = (z - mu) * jax.lax.rsqrt(var + BN_EPS) * g_ref[:] + be_ref[:]
    y = jnp.maximum(y, 0.0)
    if final:
        y_ref[:] = y
    else:
        y_ref[:] = jnp.concatenate(
            [y, jnp.zeros((NP1 - N, nout), jnp.float32)], axis=0)


@functools.cache
def _fin_call(nout, final):
    rows = N if final else NP1
    return pl.pallas_call(
        functools.partial(_fin_body, nout, final),
        out_shape=jax.ShapeDtypeStruct((rows, nout), jnp.float32),
    )


def _sc_body(dout, ch, h_hbm, asd_hbm, sd_hbm, acc_hbm, dn_hbm,
             asv, adv, dnv, idx4, rows2, pbuf, acc_sh,
             isem0, isem1, gsem0, gsem1, ssem0, ssem1):
    c = lax.axis_index("c")
    s = lax.axis_index("s")
    w = c * 16 + s
    nq = dout // 16
    k = EPT // ch
    isem = (isem0, isem1)
    gsem = (gsem0, gsem1)
    ssem = (ssem0, ssem1)

    pltpu.sync_copy(asd_hbm.at[0], asv)
    pltpu.sync_copy(asd_hbm.at[1], adv)

    def mbody(i, m):
        return jnp.maximum(m, asv[pl.ds(i * 16, 16)])
    m16 = lax.fori_loop(0, NP1 // 16, mbody, jnp.zeros((16,), jnp.float32))
    gdn = lax.GatherDimensionNumbers(
        offset_dims=(), collapsed_slice_dims=(0,), start_index_map=(0,))
    for sh in (8, 4, 2, 1):
        idx = (jnp.arange(16, dtype=jnp.int32) + sh) % 16
        perm = lax.gather(m16, idx[:, None], gdn, slice_sizes=(1,),
                          mode=lax.GatherScatterMode.PROMISE_IN_BOUNDS)
        m16 = jnp.maximum(m16, perm)
    M = m16

    zeros16 = jnp.zeros((16,), jnp.float32)

    def zdn(i, cc):
        dnv[pl.ds(i * 16, 16)] = zeros16
        return cc
    lax.fori_loop(0, NP1 // 16, zdn, 0)

    def zrow(i, cc):
        for q in range(nq):
            rows2[0, i, pl.ds(q * 16, 16)] = zeros16
        return cc
    lax.fori_loop(0, ch, zrow, 0)

    base = s * ROWS_PT
    for t in range(ROWS_PT // ch):
        pltpu.sync_copy(rows2.at[0], acc_sh.at[pl.ds(base + t * ch, ch)])
    rem = ROWS_PT % ch
    if rem:
        pltpu.sync_copy(rows2.at[0, pl.ds(0, rem)],
                        acc_sh.at[pl.ds(base + ROWS_PT - rem, rem)])
    plsc.subcore_barrier()

    def start_idx(j, m2, m4):
        pltpu.async_copy(sd_hbm.at[w, j], idx4.at[m4], isem[m2])

    def wait_idx(j, m2, m4):
        pltpu.make_async_copy(sd_hbm.at[w, j], idx4.at[m4], isem[m2]).wait()

    def start_g(b, m4):
        pltpu.async_copy(h_hbm.at[idx4.at[m4, 0]], rows2.at[b], gsem[b])

    def wait_g(b, m4):
        pltpu.make_async_copy(h_hbm.at[idx4.at[m4, 0]], rows2.at[b],
                              gsem[b]).wait()

    def start_a(b, m4):
        pltpu.async_copy(rows2.at[b], acc_sh.at[idx4.at[m4, 1]], ssem[b],
                         add=True)

    def wait_a(b, m4):
        pltpu.make_async_copy(rows2.at[b], acc_sh.at[idx4.at[m4, 1]],
                              ssem[b]).wait()

    def sub_body(j, t):
        m2, m4 = t & 1, t & 3
        om2 = 1 - m2

        @pl.when(j + 2 < k)
        def _():
            start_idx(j + 2, m2, (t + 2) & 3)

        for g in range(ch // 16):
            sidx = idx4[m4, 0, pl.ds(g * 16, 16)]
            didx = idx4[m4, 1, pl.ds(g * 16, 16)]
            u = plsc.load_gather(asv, [sidx])
            v = plsc.load_gather(adv, [didx])
            tt = u + v
            e = jnp.maximum(tt, 0.2 * tt)
            cm = M + v
            cb = jnp.maximum(cm, 0.2 * cm)
            p16 = jnp.exp(e - cb)
            pbuf[pl.ds(g * 16, 16)] = p16
            plsc.addupdate_scatter(dnv, [didx], p16)

        wait_g(m2, m4)

        def scale_body(g, cc):
            p16 = pbuf[pl.ds(g * 16, 16)]
            for l in range(16):
                pv = jnp.full((16,), p16[l], jnp.float32)
                i = g * 16 + l
                for q in range(nq):
                    rows2[m2, i, pl.ds(q * 16, 16)] = (
                        rows2[m2, i, pl.ds(q * 16, 16)] * pv)
            return cc
        lax.fori_loop(0, ch // 16, scale_body, 0)

        start_a(m2, m4)

        @pl.when(j >= 1)
        def _():
            wait_a(om2, (t + 3) & 3)

        @pl.when(j + 1 < k)
        def _():
            wait_idx(j + 1, om2, (t + 1) & 3)
            start_g(om2, (t + 1) & 3)

    start_idx(0, 0, 0)
    start_idx(1, 1, 1)
    wait_idx(0, 0, 0)
    start_g(0, 0)

    def quad(jj, cc):
        j0 = jj * 4
        for t in range(4):
            sub_body(j0 + t, t)
        return cc
    lax.fori_loop(0, k // 4, quad, 0)

    wait_a((k - 1) & 1, (k - 1) & 3)
    plsc.subcore_barrier()
    pltpu.sync_copy(acc_sh.at[pl.ds(base, ROWS_PT)],
                    acc_hbm.at[c, pl.ds(base, ROWS_PT)])
    pltpu.sync_copy(dnv, dn_hbm.at[c, s])


@functools.cache
def _sc_call(dout, ch):
    mesh = plsc.VectorSubcoreMesh(core_axis_name="c", subcore_axis_name="s")
    k = EPT // ch
    assert k % 4 == 0 and ch % 16 == 0
    return pl.kernel(
        functools.partial(_sc_body, dout, ch),
        mesh=mesh,
        compiler_params=pltpu.CompilerParams(
            needs_layout_passes=False, use_tc_tiling_on_sc=False),
        out_type=(
            jax.ShapeDtypeStruct((2, NP1, dout), jnp.float32),
            jax.ShapeDtypeStruct((2, 16, NP1), jnp.float32),
        ),
        scratch_types=[
            pltpu.VMEM((NP1,), jnp.float32),
            pltpu.VMEM((NP1,), jnp.float32),
            pltpu.VMEM((NP1,), jnp.float32),
            pltpu.VMEM((4, 2, ch), jnp.int32),
            pltpu.VMEM((2, ch, dout), jnp.float32),
            pltpu.VMEM((ch,), jnp.float32),
            pltpu.VMEM_SHARED((NP1, dout), jnp.float32),
            pltpu.SemaphoreType.DMA,
            pltpu.SemaphoreType.DMA,
            pltpu.SemaphoreType.DMA,
            pltpu.SemaphoreType.DMA,
            pltpu.SemaphoreType.DMA,
            pltpu.SemaphoreType.DMA,
        ],
    )


def _prep_edges(edge_index, ch):
    src, dst = edge_index[0], edge_index[1]
    loops = jnp.arange(N, dtype=src.dtype)
    pad = EPAD - EE
    pad_src = jnp.arange(pad, dtype=src.dtype) % N
    pad_dst = N + (jnp.arange(pad, dtype=src.dtype) % (NP1 - N))
    k = EPT // ch
    src = jnp.concatenate([src, loops, pad_src]).reshape(NW, k, ch)
    dst = jnp.concatenate([dst, loops, pad_dst]).reshape(NW, k, ch)
    return jnp.stack([src, dst], axis=2)


def _band(x, edge_index, layers):
    x = jnp.pad(x, ((0, NP1 - N), (0, 0)))
    for li, p in enumerate(layers):
        din, dout = p["W"].shape
        ch = 48 if dout > 96 else 96
        sd3 = _prep_edges(edge_index, ch)
        wa = jnp.stack([p["W"] @ p["a_src"], p["W"] @ p["a_dst"]], axis=1)
        h, asd = _mm_call(din, dout)(x, p["W"], wa)
        acc, dnp = _sc_call(dout, ch)(h, asd, sd3)
        final = li == len(layers) - 1
        x = _fin_call(dout, final)(
            acc, dnp, p["b"][None, :], p["gamma"][None, :], p["beta"][None, :])
    return x


def kernel(x_alpha, x_beta, x_theta, params,
           edge_index_alpha, edge_index_beta, edge_index_theta):
    z_a = _band(x_alpha, edge_index_alpha, params["alpha"])
    z_b = _band(x_beta, edge_index_beta, params["beta"])
    z_t = _band(x_theta, edge_index_theta, params["theta"])
    return (z_a, z_b, z_t)
